# baseline (speedup 1.0000x reference)
import math

import jax
import jax.numpy as jnp
from jax import lax
from jax.experimental import pallas as pl
from jax.experimental.pallas import tpu as pltpu
from jax.experimental.pallas import tpu_sc as plsc

N_ROWS = 20000
N_CLS = 91
FLAT = N_ROWS * 92
LROWS = FLAT // 128
BLK_R = 2875
NBLK = LROWS // BLK_R
GRP_H = 112
FULL_G = BLK_R // GRP_H
TAIL_H = BLK_R - FULL_G * GRP_H
G_PER_BLK = FULL_G + 1
CH_PER_BLK = G_PER_BLK * 128
NCH = NBLK * CH_PER_BLK
NCHV = NCH // 16
BV = NCHV // 16
CAND_CAP = 512
DEEP_CAP = 16
OUT_W = 304
NEG_INF = float("-inf")


def _chunkstats_body(x_ref, m_ref, a_ref, m2_ref):
    r = lax.rem(pl.program_id(0), NBLK)
    base = r * BLK_R
    lane128 = lax.iota(jnp.int32, 128)
    for g in range(G_PER_BLK):
        h = GRP_H if g < FULL_G else TAIL_H
        x = x_ref[0, g * GRP_H : g * GRP_H + h, :]
        m = jnp.max(x, axis=0)
        rowi = lax.broadcasted_iota(jnp.int32, (h, 128), 0)
        a = jnp.min(jnp.where(x == m[None, :], rowi, h), axis=0)
        x2 = jnp.where(rowi == a[None, :], NEG_INF, x)
        m2 = jnp.max(x2, axis=0)
        sl = pl.ds(g * 128, 128)
        m_ref[0, 0, sl] = m
        a_ref[0, 0, sl] = (base + g * GRP_H + a) * 128 + lane128
        m2_ref[0, 0, sl] = m2


def _key(x):
    b = lax.bitcast_convert_type(x, jnp.uint32)
    flip = jnp.where(x < 0.0, jnp.uint32(0xFFFFFFFF), jnp.uint32(0x80000000))
    return b ^ flip


def _unkey(u):
    flip = jnp.where(
        u >= jnp.uint32(0x80000000), jnp.uint32(0x80000000), jnp.uint32(0xFFFFFFFF)
    )
    return lax.bitcast_convert_type(u ^ flip, jnp.float32)


def _shuf(x, idx):
    dn = lax.GatherDimensionNumbers(
        offset_dims=(), collapsed_slice_dims=(0,), start_index_map=(0,)
    )
    return lax.gather(
        x,
        idx[:, None],
        dimension_numbers=dn,
        slice_sizes=(1,),
        mode=lax.GatherScatterMode.PROMISE_IN_BOUNDS,
    )


def _scalar(v):
    return jnp.max(v)


def _popcount(m):
    return _scalar(plsc.all_reduce_population_count(m))


def _fdiv(x_i32, d):
    return (x_i32.astype(jnp.float32) / jnp.float32(d)).astype(jnp.int32)


def _sc_body(amax_hbm, aidx_hbm, am2_hbm, logits_hbm, boxes_hbm, ts_hbm,
             scores_out, labels_out, boxes_out,
             amax_v, aidx_v, am2_v, gmax_v, sortk_v, sortv_v,
             dchunk_v, didx_v, deepbuf_v, boxidx_v, bidx_v, boxrows_v,
             scores_v, labels_v, boxout_v, ts_v, sem):
    nc = 2
    wid = lax.axis_index("s") * nc + lax.axis_index("c")
    lane = lax.iota(jnp.int32, 16)

    @pl.when(wid < 16)
    def _work():
        b = wid
        pltpu.sync_copy(amax_hbm.at[b], amax_v)
        pltpu.sync_copy(aidx_hbm.at[b], aidx_v)
        pltpu.sync_copy(am2_hbm.at[b], am2_v)
        pltpu.sync_copy(ts_hbm.at[b], ts_v)

        def gfold(j, _):
            def inner(k, acc):
                return jnp.maximum(acc, amax_v[pl.ds((j * 16 + k) * 16, 16)])

            acc = lax.fori_loop(0, 16, inner, jnp.full((16,), NEG_INF, jnp.float32))
            gmax_v[pl.ds(j * 16, 16)] = _key(acc)
            return 0

        lax.fori_loop(0, BV, gfold, 0)

        def count_ge(T):
            def cbody(i, acc):
                k = gmax_v[pl.ds(i * 16, 16)]
                return acc + jnp.where(k >= T, 1, 0).astype(jnp.int32)

            accv = lax.fori_loop(0, BV, cbody, jnp.zeros((16,), jnp.int32))
            return jnp.sum(accv)

        def bis_body(i, T):
            bit = 31 - i
            cand = T | (jnp.uint32(1) << bit.astype(jnp.uint32))
            c = count_ge(cand)
            return jnp.where(c >= 300, cand, T)

        t2 = lax.fori_loop(0, 24, bis_body, jnp.uint32(0))

        def zero_body(j, _):
            sortk_v[pl.ds(j * 16, 16)] = jnp.zeros((16,), jnp.uint32)
            sortv_v[pl.ds(j * 16, 16)] = jnp.zeros((16,), jnp.int32)
            return 0

        lax.fori_loop(0, CAND_CAP // 16, zero_body, 0)

        def dinit_body(j, _):
            dchunk_v[pl.ds(j * 16, 16)] = j * 16 + lane
            return 0

        lax.fori_loop(0, 2 * DEEP_CAP // 16, dinit_body, 0)

        def cmp_body(i, carry):
            off, offd = carry
            x = amax_v[pl.ds(i * 16, 16)]
            u = _key(x)
            m = u >= t2

            def do_store(carry):
                off, offd = carry
                fp = aidx_v[pl.ds(i * 16, 16)]
                offc = jnp.minimum(off, CAND_CAP - 16)
                plsc.store_compressed(sortk_v.at[pl.ds(offc, 16)], u, mask=m)
                plsc.store_compressed(sortv_v.at[pl.ds(offc, 16)], fp, mask=m)
                m2u = _key(am2_v[pl.ds(i * 16, 16)])
                md = m & (m2u >= t2)
                offdc = jnp.minimum(offd, 2 * DEEP_CAP - 16)
                plsc.store_compressed(
                    dchunk_v.at[pl.ds(offdc, 16)], i * 16 + lane, mask=md
                )
                return off + _popcount(m), offd + _popcount(md)

            return lax.cond(jnp.any(m), do_store, lambda c: c, (off, offd))

        n_cand, n_deep = lax.fori_loop(
            0, NCHV, cmp_body, (jnp.int32(0), jnp.int32(0))
        )
        n_deep = jnp.minimum(n_deep, DEEP_CAP)

        def chunk_geom(p):
            linv = p >> 7
            rv = _fdiv(linv, G_PER_BLK)
            gv = linv - rv * G_PER_BLK
            cv = p & 127
            row0 = rv * BLK_R + gv * GRP_H
            h = jnp.where(gv == FULL_G, TAIL_H, GRP_H)
            return row0, cv, h

        def didx_body(dc, _):
            zero16 = jnp.zeros((16,), jnp.int32)
            p = plsc.load_gather(dchunk_v, [zero16 + dc])
            row0, cv, h = chunk_geom(p)

            def kb(k, _):
                kk = k * 16 + lane
                rr = row0 + jnp.minimum(kk, h - 1)
                didx_v[pl.ds((dc * 8 + k) * 16, 16)] = (
                    b * FLAT + rr * 128 + cv
                )
                return 0

            lax.fori_loop(0, 8, kb, 0)
            return 0

        lax.fori_loop(0, DEEP_CAP, didx_body, 0)
        pltpu.async_copy(logits_hbm.at[didx_v], deepbuf_v, sem).wait()

        def deep_scan(dc, off):
            zero16 = jnp.zeros((16,), jnp.int32)
            p = plsc.load_gather(dchunk_v, [zero16 + dc])
            ai = plsc.load_gather(aidx_v, [p])
            row0, cv, h = chunk_geom(p)

            def kb(k, off):
                kk = k * 16 + lane
                v = deepbuf_v[pl.ds((dc * 8 + k) * 16, 16)]
                u = _key(v)
                fp = (row0 + kk) * 128 + cv
                m = (kk < h) & (u >= t2) & (fp != ai)

                def dstore(off):
                    offc = jnp.minimum(off, CAND_CAP - 16)
                    plsc.store_compressed(sortk_v.at[pl.ds(offc, 16)], u, mask=m)
                    plsc.store_compressed(sortv_v.at[pl.ds(offc, 16)], fp, mask=m)
                    return off + _popcount(m)

                return lax.cond(jnp.any(m), dstore, lambda o: o, off)

            return lax.fori_loop(0, 8, kb, off)

        n_cand = lax.fori_loop(0, n_deep, deep_scan, n_cand)

        def conv_body(j, _):
            sl = pl.ds(j * 16, 16)
            fp = sortv_v[sl]
            q = _fdiv(fp, 92)
            cls = fp - q * 92
            sortk_v[sl] = jnp.where(cls == N_CLS, jnp.uint32(0), sortk_v[sl])
            sortv_v[sl] = fp - q
            return 0

        lax.fori_loop(0, CAND_CAP // 16, conv_body, 0)

        nv = CAND_CAP // 16

        def inter_stage(ksz, j):
            jb = j // 16
            s = int(math.log2(jb)) if jb > 0 else 0

            def pair_body(t, _):
                v = ((t >> s) << (s + 1)) | (t & (jb - 1))
                p = v | jb
                ka = sortk_v[pl.ds(v * 16, 16)]
                va = sortv_v[pl.ds(v * 16, 16)]
                kb_ = sortk_v[pl.ds(p * 16, 16)]
                vb = sortv_v[pl.ds(p * 16, 16)]
                dir_asc = ((v * 16) & ksz) == 0
                lo_before = (ka > kb_) | ((ka == kb_) & (va < vb))
                swap = lo_before ^ dir_asc
                sortk_v[pl.ds(v * 16, 16)] = jnp.where(swap, kb_, ka)
                sortv_v[pl.ds(v * 16, 16)] = jnp.where(swap, vb, va)
                sortk_v[pl.ds(p * 16, 16)] = jnp.where(swap, ka, kb_)
                sortv_v[pl.ds(p * 16, 16)] = jnp.where(swap, va, vb)
                return 0

            lax.fori_loop(0, nv // 2, pair_body, 0)

        def intra_stage(ksz, j):
            pidx = lane ^ j

            def vreg_body(v, _):
                ka = sortk_v[pl.ds(v * 16, 16)]
                va = sortv_v[pl.ds(v * 16, 16)]
                kb_ = _shuf(ka, pidx)
                vb = _shuf(va, pidx)
                am_lower = (lane & j) == 0
                klo = jnp.where(am_lower, ka, kb_)
                khi = jnp.where(am_lower, kb_, ka)
                vlo = jnp.where(am_lower, va, vb)
                vhi = jnp.where(am_lower, vb, va)
                dir_asc = (((v * 16 + lane) & ksz) == 0)
                lo_before = (klo > khi) | ((klo == khi) & (vlo < vhi))
                swap = lo_before ^ dir_asc
                sortk_v[pl.ds(v * 16, 16)] = jnp.where(swap, kb_, ka)
                sortv_v[pl.ds(v * 16, 16)] = jnp.where(swap, vb, va)
                return 0

            lax.fori_loop(0, nv, vreg_body, 0)

        ksz = 2
        while ksz <= CAND_CAP:
            j = ksz // 2
            while j >= 1:
                if j >= 16:
                    inter_stage(ksz, j)
                else:
                    intra_stage(ksz, j)
                j //= 2
            ksz *= 2

        def out_body(jv, _):
            sl = pl.ds(jv * 16, 16)
            u = sortk_v[sl]
            fl = sortv_v[sl]
            x = _unkey(u)
            scores_v[sl] = 1.0 / (1.0 + jnp.exp(-x))
            br = _fdiv(fl, N_CLS)
            labels_v[sl] = fl - br * N_CLS
            boxidx_v[sl] = (b * N_ROWS + br) * 4
            return 0

        lax.fori_loop(0, OUT_W // 16, out_body, 0)

        pltpu.sync_copy(scores_v, scores_out.at[b])
        pltpu.sync_copy(labels_v, labels_out.at[b])

        def bidx_body(jv, _):
            pos = jv * 16 + lane
            base = plsc.load_gather(boxidx_v, [pos >> 2])
            bidx_v[pl.ds(jv * 16, 16)] = base + (pos & 3)
            return 0

        lax.fori_loop(0, OUT_W * 4 // 16, bidx_body, 0)
        pltpu.async_copy(boxes_hbm.at[bidx_v], boxrows_v, sem).wait()

        sc_vec = _shuf(ts_v[pl.ds(0, 16)], (lane & 1) ^ 1)

        def box_body(jv, _):
            pos = jv * 16 + lane
            cl = pos & 3
            v = boxrows_v[pl.ds(jv * 16, 16)]
            vp = plsc.load_gather(boxrows_v, [pos ^ 2])
            xy = jnp.where(cl < 2, v - 0.5 * vp, vp + 0.5 * v)
            boxout_v[pl.ds(jv * 16, 16)] = xy * sc_vec
            return 0

        lax.fori_loop(0, OUT_W * 4 // 16, box_body, 0)
        pltpu.sync_copy(boxout_v, boxes_out.at[b])


def _run_sc(amax, aidx, am2, logits_flat, boxes_flat, ts_pad):
    mesh = plsc.VectorSubcoreMesh(core_axis_name="c", subcore_axis_name="s")
    f = pl.kernel(
        _sc_body,
        mesh=mesh,
        compiler_params=pltpu.CompilerParams(needs_layout_passes=False),
        out_type=[
            jax.ShapeDtypeStruct((16, OUT_W), jnp.float32),
            jax.ShapeDtypeStruct((16, OUT_W), jnp.int32),
            jax.ShapeDtypeStruct((16, OUT_W * 4), jnp.float32),
        ],
        scratch_types=[
            pltpu.VMEM((NCH,), jnp.float32),
            pltpu.VMEM((NCH,), jnp.int32),
            pltpu.VMEM((NCH,), jnp.float32),
            pltpu.VMEM((NCHV,), jnp.uint32),
            pltpu.VMEM((CAND_CAP,), jnp.uint32),
            pltpu.VMEM((CAND_CAP,), jnp.int32),
            pltpu.VMEM((2 * DEEP_CAP,), jnp.int32),
            pltpu.VMEM((DEEP_CAP * 128,), jnp.int32),
            pltpu.VMEM((DEEP_CAP * 128,), jnp.float32),
            pltpu.VMEM((OUT_W,), jnp.int32),
            pltpu.VMEM((OUT_W * 4,), jnp.int32),
            pltpu.VMEM((OUT_W * 4,), jnp.float32),
            pltpu.VMEM((OUT_W,), jnp.float32),
            pltpu.VMEM((OUT_W,), jnp.int32),
            pltpu.VMEM((OUT_W * 4,), jnp.float32),
            pltpu.VMEM((16,), jnp.float32),
            pltpu.SemaphoreType.DMA,
        ],
    )
    return f(amax, aidx, am2, logits_flat, boxes_flat, ts_pad)


def kernel(pred_logits, pred_boxes, target_sizes):
    bs, n, c = pred_logits.shape
    xflat = pred_logits.reshape(bs * NBLK, BLK_R, 128)
    outf = jax.ShapeDtypeStruct((bs * NBLK, 1, CH_PER_BLK), jnp.float32)
    outi = jax.ShapeDtypeStruct((bs * NBLK, 1, CH_PER_BLK), jnp.int32)
    amax, aidx, am2 = pl.pallas_call(
        _chunkstats_body,
        grid=(bs * NBLK,),
        in_specs=[pl.BlockSpec((1, BLK_R, 128), lambda i: (i, 0, 0))],
        out_specs=[pl.BlockSpec((1, 1, CH_PER_BLK), lambda i: (i, 0, 0))] * 3,
        out_shape=[outf, outi, outf],
    )(xflat)
    amax = amax.reshape(bs, NCH)
    aidx = aidx.reshape(bs, NCH)
    am2 = am2.reshape(bs, NCH)

    logits_flat = pred_logits.reshape(bs * n * c)
    boxes_flat = pred_boxes.reshape(bs * n * 4)
    ts_pad = jnp.pad(target_sizes, ((0, 0), (0, 14)))

    scores_p, labels_p, boxes_p = _run_sc(
        amax, aidx, am2, logits_flat, boxes_flat, ts_pad
    )
    scores = scores_p[:, :300]
    labels = labels_p[:, :300]
    boxes = boxes_p.reshape(bs, OUT_W, 4)[:, :300]
    return scores, labels, boxes


# speedup vs baseline: 1.5320x; 1.5320x over previous
import math

import jax
import jax.numpy as jnp
from jax import lax
from jax.experimental import pallas as pl
from jax.experimental.pallas import tpu as pltpu
from jax.experimental.pallas import tpu_sc as plsc

N_ROWS = 20000
N_CLS = 91
BLK_R = 2000
NBLK = N_ROWS // BLK_R
GRP_H = 200
G_PER_BLK = BLK_R // GRP_H
CH_PER_BLK = G_PER_BLK * 128
NCH = NBLK * CH_PER_BLK
NCHV = NCH // 16
BV = NCHV // 16
CAND_CAP = 512
DEEP_CAP = 24
DLIST = 48
OUT_W = 304
NEG_INF = float("-inf")


def _chunkstats_body(x_ref, m_ref, a_ref, m2_ref):
    r = lax.rem(pl.program_id(0), NBLK)
    base = r * BLK_R
    neg36 = jnp.full((36,), NEG_INF, jnp.float32)
    zero36 = jnp.zeros((36,), jnp.int32)
    lane92 = lax.iota(jnp.int32, 92)
    for g in range(G_PER_BLK):
        x = x_ref[0, g * GRP_H : (g + 1) * GRP_H, :]
        m = jnp.max(x, axis=0)
        rowi = lax.broadcasted_iota(jnp.int32, (GRP_H, 92), 0)
        a = jnp.min(jnp.where(x == m[None, :], rowi, GRP_H), axis=0)
        x2 = jnp.where(rowi == a[None, :], NEG_INF, x)
        m2 = jnp.max(x2, axis=0)
        m = jnp.where(lane92 >= N_CLS, NEG_INF, m)
        m2 = jnp.where(lane92 >= N_CLS, NEG_INF, m2)
        fl = (base + g * GRP_H + a) * N_CLS + lane92
        sl = pl.ds(g * 128, 128)
        m_ref[0, 0, sl] = jnp.concatenate([m, neg36])
        a_ref[0, 0, sl] = jnp.concatenate([fl, zero36])
        m2_ref[0, 0, sl] = jnp.concatenate([m2, neg36])


def _key(x):
    b = lax.bitcast_convert_type(x, jnp.uint32)
    flip = jnp.where(x < 0.0, jnp.uint32(0xFFFFFFFF), jnp.uint32(0x80000000))
    return b ^ flip


def _unkey(u):
    flip = jnp.where(
        u >= jnp.uint32(0x80000000), jnp.uint32(0x80000000), jnp.uint32(0xFFFFFFFF)
    )
    return lax.bitcast_convert_type(u ^ flip, jnp.float32)


def _shuf(x, idx):
    dn = lax.GatherDimensionNumbers(
        offset_dims=(), collapsed_slice_dims=(0,), start_index_map=(0,)
    )
    return lax.gather(
        x,
        idx[:, None],
        dimension_numbers=dn,
        slice_sizes=(1,),
        mode=lax.GatherScatterMode.PROMISE_IN_BOUNDS,
    )


def _scalar(v):
    return jnp.max(v)


def _popcount(m):
    return _scalar(plsc.all_reduce_population_count(m))


def _fdiv(x_i32, d):
    return (x_i32.astype(jnp.float32) / jnp.float32(d)).astype(jnp.int32)


def _sc_body(amax_hbm, aidx_hbm, am2_hbm, logits_hbm, boxes_hbm, ts_hbm,
             scores_out, labels_out, boxes_out,
             amax_v, aidx_v, am2_v, gmax_v, sortk_v, sortv_v,
             dchunk_v, deepstage_v, boxidx_v, bidx_v, boxrows_v,
             scores_v, labels_v, boxout_v, ts_v, sem):
    nc = 2
    wid = lax.axis_index("s") * nc + lax.axis_index("c")
    lane = lax.iota(jnp.int32, 16)

    @pl.when(wid < 16)
    def _work():
        b = wid
        pltpu.sync_copy(amax_hbm.at[b], amax_v)
        pltpu.sync_copy(aidx_hbm.at[b], aidx_v)
        pltpu.sync_copy(am2_hbm.at[b], am2_v)
        pltpu.sync_copy(ts_hbm.at[b], ts_v)

        def gfold(j, _):
            def inner(k, acc):
                return jnp.maximum(acc, amax_v[pl.ds((j * 16 + k) * 16, 16)])

            acc = lax.fori_loop(0, 16, inner, jnp.full((16,), NEG_INF, jnp.float32))
            gmax_v[pl.ds(j * 16, 16)] = _key(acc)
            return 0

        lax.fori_loop(0, BV, gfold, 0)

        def count_ge(T):
            def cbody(i, acc):
                k = gmax_v[pl.ds(i * 16, 16)]
                return acc + jnp.where(k >= T, 1, 0).astype(jnp.int32)

            accv = lax.fori_loop(0, BV, cbody, jnp.zeros((16,), jnp.int32))
            return jnp.sum(accv)

        def bis_body(i, T):
            bit = 31 - i
            cand = T | (jnp.uint32(1) << bit.astype(jnp.uint32))
            c = count_ge(cand)
            return jnp.where(c >= 300, cand, T)

        t2 = lax.fori_loop(0, 24, bis_body, jnp.uint32(0))

        def zero_body(j, _):
            sortk_v[pl.ds(j * 16, 16)] = jnp.zeros((16,), jnp.uint32)
            sortv_v[pl.ds(j * 16, 16)] = jnp.zeros((16,), jnp.int32)
            return 0

        lax.fori_loop(0, CAND_CAP // 16, zero_body, 0)

        def dinit_body(j, _):
            dchunk_v[pl.ds(j * 16, 16)] = j * 16 + lane
            return 0

        lax.fori_loop(0, DLIST // 16, dinit_body, 0)

        def cmp_body(i, carry):
            off, offd = carry
            x = amax_v[pl.ds(i * 16, 16)]
            u = _key(x)
            m = u >= t2

            def do_store(carry):
                off, offd = carry
                fp = aidx_v[pl.ds(i * 16, 16)]
                offc = jnp.minimum(off, CAND_CAP - 16)
                plsc.store_compressed(sortk_v.at[pl.ds(offc, 16)], u, mask=m)
                plsc.store_compressed(sortv_v.at[pl.ds(offc, 16)], fp, mask=m)
                m2u = _key(am2_v[pl.ds(i * 16, 16)])
                md = m & (m2u >= t2)
                offdc = jnp.minimum(offd, DLIST - 16)
                plsc.store_compressed(
                    dchunk_v.at[pl.ds(offdc, 16)], i * 16 + lane, mask=md
                )
                return off + _popcount(m), offd + _popcount(md)

            return lax.cond(jnp.any(m), do_store, lambda c: c, (off, offd))

        n_cand, n_deep = lax.fori_loop(
            0, NCHV, cmp_body, (jnp.int32(0), jnp.int32(0))
        )
        n_deep = jnp.minimum(n_deep, DEEP_CAP)

        def deep_scan(dc, off):
            zero16 = jnp.zeros((16,), jnp.int32)
            p = plsc.load_gather(dchunk_v, [zero16 + dc])
            ai = plsc.load_gather(aidx_v, [p])
            cv = p & 127
            row0 = (p >> 7) * GRP_H
            row0_s = jnp.max(row0)
            src0 = pl.multiple_of(b * N_ROWS + row0_s, 8)
            pltpu.sync_copy(logits_hbm.at[pl.ds(src0, GRP_H)], deepstage_v)

            def kb(k, off):
                kk = k * 16 + lane
                rr = jnp.minimum(kk, GRP_H - 1)
                v = plsc.load_gather(deepstage_v, [rr, cv])
                u = _key(v)
                fl = (row0 + kk) * N_CLS + cv
                m = (kk < GRP_H) & (u >= t2) & (fl != ai)

                def dstore(off):
                    offc = jnp.minimum(off, CAND_CAP - 16)
                    plsc.store_compressed(sortk_v.at[pl.ds(offc, 16)], u, mask=m)
                    plsc.store_compressed(sortv_v.at[pl.ds(offc, 16)], fl, mask=m)
                    return off + _popcount(m)

                return lax.cond(jnp.any(m), dstore, lambda o: o, off)

            return lax.fori_loop(0, (GRP_H + 15) // 16, kb, off)

        n_cand = lax.fori_loop(0, n_deep, deep_scan, n_cand)

        nv = CAND_CAP // 16

        def inter_stage(ksz, j):
            jb = j // 16
            s = int(math.log2(jb)) if jb > 0 else 0

            def pair_body(t, _):
                v = ((t >> s) << (s + 1)) | (t & (jb - 1))
                p = v | jb
                ka = sortk_v[pl.ds(v * 16, 16)]
                va = sortv_v[pl.ds(v * 16, 16)]
                kb_ = sortk_v[pl.ds(p * 16, 16)]
                vb = sortv_v[pl.ds(p * 16, 16)]
                dir_asc = ((v * 16) & ksz) == 0
                lo_before = (ka > kb_) | ((ka == kb_) & (va < vb))
                swap = lo_before ^ dir_asc
                sortk_v[pl.ds(v * 16, 16)] = jnp.where(swap, kb_, ka)
                sortv_v[pl.ds(v * 16, 16)] = jnp.where(swap, vb, va)
                sortk_v[pl.ds(p * 16, 16)] = jnp.where(swap, ka, kb_)
                sortv_v[pl.ds(p * 16, 16)] = jnp.where(swap, va, vb)
                return 0

            lax.fori_loop(0, nv // 2, pair_body, 0)

        def intra_stage(ksz, j):
            pidx = lane ^ j

            def vreg_body(v, _):
                ka = sortk_v[pl.ds(v * 16, 16)]
                va = sortv_v[pl.ds(v * 16, 16)]
                kb_ = _shuf(ka, pidx)
                vb = _shuf(va, pidx)
                am_lower = (lane & j) == 0
                klo = jnp.where(am_lower, ka, kb_)
                khi = jnp.where(am_lower, kb_, ka)
                vlo = jnp.where(am_lower, va, vb)
                vhi = jnp.where(am_lower, vb, va)
                dir_asc = (((v * 16 + lane) & ksz) == 0)
                lo_before = (klo > khi) | ((klo == khi) & (vlo < vhi))
                swap = lo_before ^ dir_asc
                sortk_v[pl.ds(v * 16, 16)] = jnp.where(swap, kb_, ka)
                sortv_v[pl.ds(v * 16, 16)] = jnp.where(swap, vb, va)
                return 0

            lax.fori_loop(0, nv, vreg_body, 0)

        ksz = 2
        while ksz <= CAND_CAP:
            j = ksz // 2
            while j >= 1:
                if j >= 16:
                    inter_stage(ksz, j)
                else:
                    intra_stage(ksz, j)
                j //= 2
            ksz *= 2

        def out_body(jv, _):
            sl = pl.ds(jv * 16, 16)
            u = sortk_v[sl]
            fl = sortv_v[sl]
            x = _unkey(u)
            scores_v[sl] = 1.0 / (1.0 + jnp.exp(-x))
            br = _fdiv(fl, N_CLS)
            labels_v[sl] = fl - br * N_CLS
            boxidx_v[sl] = (b * N_ROWS + br) * 4
            return 0

        lax.fori_loop(0, OUT_W // 16, out_body, 0)

        pltpu.sync_copy(scores_v, scores_out.at[b])
        pltpu.sync_copy(labels_v, labels_out.at[b])

        def bidx_body(jv, _):
            pos = jv * 16 + lane
            base = plsc.load_gather(boxidx_v, [pos >> 2])
            bidx_v[pl.ds(jv * 16, 16)] = base + (pos & 3)
            return 0

        lax.fori_loop(0, OUT_W * 4 // 16, bidx_body, 0)
        pltpu.async_copy(boxes_hbm.at[bidx_v], boxrows_v, sem).wait()

        sc_vec = _shuf(ts_v[pl.ds(0, 16)], (lane & 1) ^ 1)

        def box_body(jv, _):
            pos = jv * 16 + lane
            cl = pos & 3
            v = boxrows_v[pl.ds(jv * 16, 16)]
            vp = plsc.load_gather(boxrows_v, [pos ^ 2])
            xy = jnp.where(cl < 2, v - 0.5 * vp, vp + 0.5 * v)
            boxout_v[pl.ds(jv * 16, 16)] = xy * sc_vec
            return 0

        lax.fori_loop(0, OUT_W * 4 // 16, box_body, 0)
        pltpu.sync_copy(boxout_v, boxes_out.at[b])


def _run_sc(amax, aidx, am2, logits_flat, boxes_flat, ts_pad):
    mesh = plsc.VectorSubcoreMesh(core_axis_name="c", subcore_axis_name="s")
    f = pl.kernel(
        _sc_body,
        mesh=mesh,
        compiler_params=pltpu.CompilerParams(needs_layout_passes=False),
        out_type=[
            jax.ShapeDtypeStruct((16, OUT_W), jnp.float32),
            jax.ShapeDtypeStruct((16, OUT_W), jnp.int32),
            jax.ShapeDtypeStruct((16, OUT_W * 4), jnp.float32),
        ],
        scratch_types=[
            pltpu.VMEM((NCH,), jnp.float32),
            pltpu.VMEM((NCH,), jnp.int32),
            pltpu.VMEM((NCH,), jnp.float32),
            pltpu.VMEM((NCHV,), jnp.uint32),
            pltpu.VMEM((CAND_CAP,), jnp.uint32),
            pltpu.VMEM((CAND_CAP,), jnp.int32),
            pltpu.VMEM((DLIST,), jnp.int32),
            pltpu.VMEM((GRP_H, 92), jnp.float32),
            pltpu.VMEM((OUT_W,), jnp.int32),
            pltpu.VMEM((OUT_W * 4,), jnp.int32),
            pltpu.VMEM((OUT_W * 4,), jnp.float32),
            pltpu.VMEM((OUT_W,), jnp.float32),
            pltpu.VMEM((OUT_W,), jnp.int32),
            pltpu.VMEM((OUT_W * 4,), jnp.float32),
            pltpu.VMEM((16,), jnp.float32),
            pltpu.SemaphoreType.DMA,
        ],
    )
    return f(amax, aidx, am2, logits_flat, boxes_flat, ts_pad)


def kernel(pred_logits, pred_boxes, target_sizes):
    bs, n, c = pred_logits.shape
    outf = jax.ShapeDtypeStruct((bs * NBLK, 1, CH_PER_BLK), jnp.float32)
    outi = jax.ShapeDtypeStruct((bs * NBLK, 1, CH_PER_BLK), jnp.int32)
    amax, aidx, am2 = pl.pallas_call(
        _chunkstats_body,
        grid=(bs * NBLK,),
        in_specs=[
            pl.BlockSpec((1, BLK_R, c), lambda i: (i, 0, 0))
        ],
        out_specs=[pl.BlockSpec((1, 1, CH_PER_BLK), lambda i: (i, 0, 0))] * 3,
        out_shape=[outf, outi, outf],
    )(pred_logits.reshape(bs * NBLK, BLK_R, c))
    amax = amax.reshape(bs, NCH)
    aidx = aidx.reshape(bs, NCH)
    am2 = am2.reshape(bs, NCH)

    logits_flat = pred_logits.reshape(bs * n, c)
    boxes_flat = pred_boxes.reshape(bs * n * 4)
    ts_pad = jnp.pad(target_sizes, ((0, 0), (0, 14)))

    scores_p, labels_p, boxes_p = _run_sc(
        amax, aidx, am2, logits_flat, boxes_flat, ts_pad
    )
    scores = scores_p[:, :300]
    labels = labels_p[:, :300]
    boxes = boxes_p.reshape(bs, OUT_W, 4)[:, :300]
    return scores, labels, boxes


# speedup vs baseline: 5.7816x; 3.7739x over previous
import math

import jax
import jax.numpy as jnp
from jax import lax
from jax.experimental import pallas as pl
from jax.experimental.pallas import tpu as pltpu
from jax.experimental.pallas import tpu_sc as plsc

N_ROWS = 20000
N_CLS = 91
NP = 20480
HALF = NP // 2
HV = HALF // 16
FOLD = 20
BV = 2 * (HV // FOLD)
CAND_CAP = 512
DEEP_CAP = 24
DLIST = 48
OUT_W = 304
NEG_INF = float("-inf")


def _rowstats_body(x_ref, m_ref, a_ref, m2_ref):
    c = pl.program_id(0)

    @pl.when(c == 0)
    def _init():
        x = x_ref[0]
        m_ref[:, :N_ROWS] = x
        m_ref[:, N_ROWS:] = jnp.full((16, NP - N_ROWS), NEG_INF, jnp.float32)
        a_ref[...] = jnp.zeros((16, NP), jnp.int32)
        m2_ref[...] = jnp.full((16, NP), NEG_INF, jnp.float32)

    @pl.when((c > 0) & (c < N_CLS))
    def _upd():
        x = x_ref[0]
        mo = m_ref[:, :N_ROWS]
        better = x > mo
        m_ref[:, :N_ROWS] = jnp.maximum(mo, x)
        a_ref[:, :N_ROWS] = jnp.where(better, c, a_ref[:, :N_ROWS])
        m2_ref[:, :N_ROWS] = jnp.maximum(
            m2_ref[:, :N_ROWS], jnp.where(better, mo, x)
        )


def _key(x):
    b = lax.bitcast_convert_type(x, jnp.uint32)
    flip = jnp.where(x < 0.0, jnp.uint32(0xFFFFFFFF), jnp.uint32(0x80000000))
    return b ^ flip


def _unkey(u):
    flip = jnp.where(
        u >= jnp.uint32(0x80000000), jnp.uint32(0x80000000), jnp.uint32(0xFFFFFFFF)
    )
    return lax.bitcast_convert_type(u ^ flip, jnp.float32)


def _shuf(x, idx):
    dn = lax.GatherDimensionNumbers(
        offset_dims=(), collapsed_slice_dims=(0,), start_index_map=(0,)
    )
    return lax.gather(
        x,
        idx[:, None],
        dimension_numbers=dn,
        slice_sizes=(1,),
        mode=lax.GatherScatterMode.PROMISE_IN_BOUNDS,
    )


def _scalar(v):
    return jnp.max(v)


def _popcount(m):
    return _scalar(plsc.all_reduce_population_count(m))


def _fdiv(x_i32, d):
    return (x_i32.astype(jnp.float32) / jnp.float32(d)).astype(jnp.int32)


def _sc_body(rm_hbm, am_hbm, m2_hbm, lt_hbm, boxes_hbm, ts_hbm,
             scores_out, labels_out, boxes_out,
             rm_v, am_v, m2_v, gmax_v, sortk_v, sortv_v,
             dchunk_v, dai_v, deepstage_v, boxstage_v, boxidx_v,
             scores_v, labels_v, boxout_v, ts_v, sem):
    nc = 2
    wid = lax.axis_index("s") * nc + lax.axis_index("c")
    lane = lax.iota(jnp.int32, 16)

    @pl.when(wid < 16)
    def _work():
        b = wid
        pltpu.sync_copy(ts_hbm.at[b], ts_v)

        def stage_half(h):
            sl = pl.ds(h * HALF, HALF)
            pltpu.sync_copy(rm_hbm.at[b].at[sl], rm_v)
            pltpu.sync_copy(am_hbm.at[b].at[sl], am_v)
            pltpu.sync_copy(m2_hbm.at[b].at[sl], m2_v)

        def fold_half(h):
            def gfold(j, _):
                def inner(k, acc):
                    return jnp.maximum(acc, rm_v[pl.ds((j * FOLD + k) * 16, 16)])

                acc = lax.fori_loop(
                    0, FOLD, inner, jnp.full((16,), NEG_INF, jnp.float32)
                )
                gmax_v[pl.ds((h * (BV // 2) + j) * 16, 16)] = _key(acc)
                return 0

            lax.fori_loop(0, BV // 2, gfold, 0)

        stage_half(0)
        fold_half(0)
        stage_half(1)
        fold_half(1)

        def count_ge(T):
            def cbody(i, acc):
                k = gmax_v[pl.ds(i * 16, 16)]
                return acc + jnp.where(k >= T, 1, 0).astype(jnp.int32)

            accv = lax.fori_loop(0, BV, cbody, jnp.zeros((16,), jnp.int32))
            return jnp.sum(accv)

        def bis_body(i, T):
            bit = 31 - i
            cand = T | (jnp.uint32(1) << bit.astype(jnp.uint32))
            c = count_ge(cand)
            return jnp.where(c >= 300, cand, T)

        t2 = lax.fori_loop(0, 24, bis_body, jnp.uint32(0))

        def zero_body(j, _):
            sortk_v[pl.ds(j * 16, 16)] = jnp.zeros((16,), jnp.uint32)
            sortv_v[pl.ds(j * 16, 16)] = jnp.zeros((16,), jnp.int32)
            return 0

        lax.fori_loop(0, CAND_CAP // 16, zero_body, 0)

        def dinit_body(j, _):
            dchunk_v[pl.ds(j * 16, 16)] = j * 16 + lane
            dai_v[pl.ds(j * 16, 16)] = jnp.full((16,), -1, jnp.int32)
            return 0

        lax.fori_loop(0, DLIST // 16, dinit_body, 0)

        def compact_half(h, carry):
            def cmp_body(i, carry):
                off, offd = carry
                x = rm_v[pl.ds(i * 16, 16)]
                u = _key(x)
                m = u >= t2

                def do_store(carry):
                    off, offd = carry
                    am = am_v[pl.ds(i * 16, 16)]
                    row = h * HALF + i * 16 + lane
                    fl = row * N_CLS + am
                    offc = jnp.minimum(off, CAND_CAP - 16)
                    plsc.store_compressed(sortk_v.at[pl.ds(offc, 16)], u, mask=m)
                    plsc.store_compressed(sortv_v.at[pl.ds(offc, 16)], fl, mask=m)
                    m2u = _key(m2_v[pl.ds(i * 16, 16)])
                    md = m & (m2u >= t2)
                    offdc = jnp.minimum(offd, DLIST - 16)
                    plsc.store_compressed(
                        dchunk_v.at[pl.ds(offdc, 16)], row, mask=md
                    )
                    plsc.store_compressed(dai_v.at[pl.ds(offdc, 16)], fl, mask=md)
                    return off + _popcount(m), offd + _popcount(md)

                return lax.cond(jnp.any(m), do_store, lambda cr: cr, carry)

            return lax.fori_loop(0, HV, cmp_body, carry)

        carry = compact_half(1, (jnp.int32(0), jnp.int32(0)))
        stage_half(0)
        n_cand, n_deep = compact_half(0, carry)
        n_deep = jnp.minimum(n_deep, DEEP_CAP)

        def deep_scan(dc, off):
            zero16 = jnp.zeros((16,), jnp.int32)
            r_vec = plsc.load_gather(dchunk_v, [zero16 + dc])
            ai = plsc.load_gather(dai_v, [zero16 + dc])
            r_s = jnp.max(r_vec)
            r128 = jnp.minimum((r_s >> 7) << 7, N_ROWS - 128)
            col = r_s - r128
            src0 = pl.multiple_of(r128, 128)
            pltpu.sync_copy(lt_hbm.at[:, b, pl.ds(src0, 128)], deepstage_v)

            def kb(k, off):
                cls = k * 16 + lane
                v = plsc.load_gather(
                    deepstage_v, [jnp.minimum(cls, N_CLS), zero16 + col]
                )
                u = _key(v)
                fl = r_vec * N_CLS + cls
                m = (cls < N_CLS) & (u >= t2) & (fl != ai)

                def dstore(off):
                    offc = jnp.minimum(off, CAND_CAP - 16)
                    plsc.store_compressed(sortk_v.at[pl.ds(offc, 16)], u, mask=m)
                    plsc.store_compressed(sortv_v.at[pl.ds(offc, 16)], fl, mask=m)
                    return off + _popcount(m)

                return lax.cond(jnp.any(m), dstore, lambda o: o, off)

            return lax.fori_loop(0, 6, kb, off)

        n_cand = lax.fori_loop(0, n_deep, deep_scan, n_cand)

        nv = CAND_CAP // 16

        def inter_stage(ksz, j):
            jb = j // 16
            s = int(math.log2(jb)) if jb > 0 else 0

            def pair_body(t, _):
                v = ((t >> s) << (s + 1)) | (t & (jb - 1))
                p = v | jb
                ka = sortk_v[pl.ds(v * 16, 16)]
                va = sortv_v[pl.ds(v * 16, 16)]
                kb_ = sortk_v[pl.ds(p * 16, 16)]
                vb = sortv_v[pl.ds(p * 16, 16)]
                dir_asc = ((v * 16) & ksz) == 0
                lo_before = (ka > kb_) | ((ka == kb_) & (va < vb))
                swap = lo_before ^ dir_asc
                sortk_v[pl.ds(v * 16, 16)] = jnp.where(swap, kb_, ka)
                sortv_v[pl.ds(v * 16, 16)] = jnp.where(swap, vb, va)
                sortk_v[pl.ds(p * 16, 16)] = jnp.where(swap, ka, kb_)
                sortv_v[pl.ds(p * 16, 16)] = jnp.where(swap, va, vb)
                return 0

            lax.fori_loop(0, nv // 2, pair_body, 0)

        def intra_stage(ksz, j):
            pidx = lane ^ j

            def vreg_body(v, _):
                ka = sortk_v[pl.ds(v * 16, 16)]
                va = sortv_v[pl.ds(v * 16, 16)]
                kb_ = _shuf(ka, pidx)
                vb = _shuf(va, pidx)
                am_lower = (lane & j) == 0
                klo = jnp.where(am_lower, ka, kb_)
                khi = jnp.where(am_lower, kb_, ka)
                vlo = jnp.where(am_lower, va, vb)
                vhi = jnp.where(am_lower, vb, va)
                dir_asc = (((v * 16 + lane) & ksz) == 0)
                lo_before = (klo > khi) | ((klo == khi) & (vlo < vhi))
                swap = lo_before ^ dir_asc
                sortk_v[pl.ds(v * 16, 16)] = jnp.where(swap, kb_, ka)
                sortv_v[pl.ds(v * 16, 16)] = jnp.where(swap, vb, va)
                return 0

            lax.fori_loop(0, nv, vreg_body, 0)

        ksz = 2
        while ksz <= CAND_CAP:
            j = ksz // 2
            while j >= 1:
                if j >= 16:
                    inter_stage(ksz, j)
                else:
                    intra_stage(ksz, j)
                j //= 2
            ksz *= 2

        def out_body(jv, _):
            sl = pl.ds(jv * 16, 16)
            u = sortk_v[sl]
            fl = sortv_v[sl]
            x = _unkey(u)
            scores_v[sl] = 1.0 / (1.0 + jnp.exp(-x))
            br = _fdiv(fl, N_CLS)
            labels_v[sl] = fl - br * N_CLS
            boxidx_v[sl] = br
            return 0

        lax.fori_loop(0, OUT_W // 16, out_body, 0)

        pltpu.sync_copy(scores_v, scores_out.at[b])
        pltpu.sync_copy(labels_v, labels_out.at[b])

        pltpu.sync_copy(boxes_hbm.at[b], boxstage_v)
        sc_vec = _shuf(ts_v[pl.ds(0, 16)], (lane & 1) ^ 1)

        def box_body(jv, _):
            pos = jv * 16 + lane
            q = pos & 3
            br = plsc.load_gather(boxidx_v, [pos >> 2])
            v = plsc.load_gather(boxstage_v, [q, br])
            vp = plsc.load_gather(boxstage_v, [q ^ 2, br])
            xy = jnp.where(q < 2, v - 0.5 * vp, vp + 0.5 * v)
            boxout_v[pl.ds(jv * 16, 16)] = xy * sc_vec
            return 0

        lax.fori_loop(0, OUT_W * 4 // 16, box_body, 0)
        pltpu.sync_copy(boxout_v, boxes_out.at[b])


def _run_sc(rm, am, m2, lt, boxes_t, ts_pad):
    mesh = plsc.VectorSubcoreMesh(core_axis_name="c", subcore_axis_name="s")
    f = pl.kernel(
        _sc_body,
        mesh=mesh,
        compiler_params=pltpu.CompilerParams(needs_layout_passes=False),
        out_type=[
            jax.ShapeDtypeStruct((16, OUT_W), jnp.float32),
            jax.ShapeDtypeStruct((16, OUT_W), jnp.int32),
            jax.ShapeDtypeStruct((16, OUT_W * 4), jnp.float32),
        ],
        scratch_types=[
            pltpu.VMEM((HALF,), jnp.float32),
            pltpu.VMEM((HALF,), jnp.int32),
            pltpu.VMEM((HALF,), jnp.float32),
            pltpu.VMEM((BV * 16,), jnp.uint32),
            pltpu.VMEM((CAND_CAP,), jnp.uint32),
            pltpu.VMEM((CAND_CAP,), jnp.int32),
            pltpu.VMEM((DLIST,), jnp.int32),
            pltpu.VMEM((DLIST,), jnp.int32),
            pltpu.VMEM((92, 128), jnp.float32),
            pltpu.VMEM((4, N_ROWS), jnp.float32),
            pltpu.VMEM((OUT_W,), jnp.int32),
            pltpu.VMEM((OUT_W,), jnp.float32),
            pltpu.VMEM((OUT_W,), jnp.int32),
            pltpu.VMEM((OUT_W * 4,), jnp.float32),
            pltpu.VMEM((16,), jnp.float32),
            pltpu.SemaphoreType.DMA,
        ],
    )
    return f(rm, am, m2, lt, boxes_t, ts_pad)


def kernel(pred_logits, pred_boxes, target_sizes):
    bs, n, c = pred_logits.shape
    lt = jnp.transpose(pred_logits, (2, 0, 1))
    rm, am, m2 = pl.pallas_call(
        _rowstats_body,
        grid=(c,),
        in_specs=[pl.BlockSpec((1, bs, n), lambda cc: (cc, 0, 0))],
        out_specs=[pl.BlockSpec((bs, NP), lambda cc: (0, 0))] * 3,
        out_shape=[
            jax.ShapeDtypeStruct((bs, NP), jnp.float32),
            jax.ShapeDtypeStruct((bs, NP), jnp.int32),
            jax.ShapeDtypeStruct((bs, NP), jnp.float32),
        ],
    )(lt)

    boxes_t = jnp.transpose(pred_boxes, (0, 2, 1))
    ts_pad = jnp.pad(target_sizes, ((0, 0), (0, 14)))

    scores_p, labels_p, boxes_p = _run_sc(rm, am, m2, lt, boxes_t, ts_pad)
    scores = scores_p[:, :300]
    labels = labels_p[:, :300]
    boxes = boxes_p.reshape(bs, OUT_W, 4)[:, :300]
    return scores, labels, boxes


# speedup vs baseline: 6.8980x; 1.1931x over previous
import math

import jax
import jax.numpy as jnp
from jax import lax
from jax.experimental import pallas as pl
from jax.experimental.pallas import tpu as pltpu
from jax.experimental.pallas import tpu_sc as plsc

N_ROWS = 20000
N_CLS = 91
NP = 20480
HALF = NP // 2
HV = HALF // 16
FOLD = 20
BV = 2 * (HV // FOLD)
CAND_CAP = 512
DEEP_CAP = 24
DLIST = 48
OUT_W = 304
NEG_INF = float("-inf")


def _rowstats_body(x_ref, m_ref, a_ref, m2_ref):
    pid = pl.program_id(0)
    c0 = pid * 4
    x0 = x_ref[0]
    x1 = x_ref[1]
    x2 = x_ref[2]
    x3 = jnp.where(c0 + 3 >= N_CLS, NEG_INF, x_ref[3])
    m01 = jnp.maximum(x0, x1)
    i01 = jnp.where(x1 > x0, c0 + 1, c0)
    n01 = jnp.minimum(x0, x1)
    m23 = jnp.maximum(x2, x3)
    i23 = jnp.where(x3 > x2, c0 + 3, c0 + 2)
    n23 = jnp.minimum(x2, x3)
    s1 = jnp.maximum(m01, m23)
    i4 = jnp.where(m23 > m01, i23, i01)
    s2 = jnp.maximum(jnp.minimum(m01, m23), jnp.maximum(n01, n23))

    @pl.when(pid == 0)
    def _init():
        m_ref[:, :N_ROWS] = s1
        m_ref[:, N_ROWS:] = jnp.full((16, NP - N_ROWS), NEG_INF, jnp.float32)
        a_ref[:, :N_ROWS] = i4
        a_ref[:, N_ROWS:] = jnp.zeros((16, NP - N_ROWS), jnp.int32)
        m2_ref[:, :N_ROWS] = s2
        m2_ref[:, N_ROWS:] = jnp.full((16, NP - N_ROWS), NEG_INF, jnp.float32)

    @pl.when(pid > 0)
    def _upd():
        mo = m_ref[:, :N_ROWS]
        m2o = m2_ref[:, :N_ROWS]
        better = s1 > mo
        m_ref[:, :N_ROWS] = jnp.maximum(mo, s1)
        a_ref[:, :N_ROWS] = jnp.where(better, i4, a_ref[:, :N_ROWS])
        m2_ref[:, :N_ROWS] = jnp.maximum(
            jnp.minimum(mo, s1), jnp.maximum(m2o, s2)
        )


def _key(x):
    b = lax.bitcast_convert_type(x, jnp.uint32)
    flip = jnp.where(x < 0.0, jnp.uint32(0xFFFFFFFF), jnp.uint32(0x80000000))
    return b ^ flip


def _unkey(u):
    flip = jnp.where(
        u >= jnp.uint32(0x80000000), jnp.uint32(0x80000000), jnp.uint32(0xFFFFFFFF)
    )
    return lax.bitcast_convert_type(u ^ flip, jnp.float32)


def _shuf(x, idx):
    dn = lax.GatherDimensionNumbers(
        offset_dims=(), collapsed_slice_dims=(0,), start_index_map=(0,)
    )
    return lax.gather(
        x,
        idx[:, None],
        dimension_numbers=dn,
        slice_sizes=(1,),
        mode=lax.GatherScatterMode.PROMISE_IN_BOUNDS,
    )


def _scalar(v):
    return jnp.max(v)


def _popcount(m):
    return _scalar(plsc.all_reduce_population_count(m))


def _fdiv(x_i32, d):
    return (x_i32.astype(jnp.float32) / jnp.float32(d)).astype(jnp.int32)


def _sc_body(rm_hbm, am_hbm, m2_hbm, lt_hbm, boxes_hbm, ts_hbm,
             scores_out, labels_out, boxes_out,
             rm_v, am_v, m2_v, gmax_v, sortk_v, sortv_v,
             dchunk_v, dai_v, deepstage_v, boxstage_v, boxidx_v,
             scores_v, labels_v, boxout_v, ts_v, sem):
    nc = 2
    wid = lax.axis_index("s") * nc + lax.axis_index("c")
    lane = lax.iota(jnp.int32, 16)

    @pl.when(wid < 16)
    def _work():
        b = wid
        pltpu.sync_copy(ts_hbm.at[b], ts_v)

        def stage_half(h):
            sl = pl.ds(h * HALF, HALF)
            pltpu.sync_copy(rm_hbm.at[b].at[sl], rm_v)
            pltpu.sync_copy(am_hbm.at[b].at[sl], am_v)
            pltpu.sync_copy(m2_hbm.at[b].at[sl], m2_v)

        def fold_half(h):
            def gfold(j, _):
                def inner(k, acc):
                    return jnp.maximum(acc, rm_v[pl.ds((j * FOLD + k) * 16, 16)])

                acc = lax.fori_loop(
                    0, FOLD, inner, jnp.full((16,), NEG_INF, jnp.float32)
                )
                gmax_v[pl.ds((h * (BV // 2) + j) * 16, 16)] = _key(acc)
                return 0

            lax.fori_loop(0, BV // 2, gfold, 0)

        stage_half(0)
        fold_half(0)
        stage_half(1)
        fold_half(1)

        def count_ge(T):
            def cbody(i, acc):
                k = gmax_v[pl.ds(i * 16, 16)]
                return acc + jnp.where(k >= T, 1, 0).astype(jnp.int32)

            accv = lax.fori_loop(0, BV, cbody, jnp.zeros((16,), jnp.int32))
            return jnp.sum(accv)

        def bis_body(i, T):
            bit = 31 - i
            cand = T | (jnp.uint32(1) << bit.astype(jnp.uint32))
            c = count_ge(cand)
            return jnp.where(c >= 300, cand, T)

        t2 = lax.fori_loop(0, 24, bis_body, jnp.uint32(0))

        def zero_body(j, _):
            sortk_v[pl.ds(j * 16, 16)] = jnp.zeros((16,), jnp.uint32)
            sortv_v[pl.ds(j * 16, 16)] = jnp.zeros((16,), jnp.int32)
            return 0

        lax.fori_loop(0, CAND_CAP // 16, zero_body, 0)

        def dinit_body(j, _):
            dchunk_v[pl.ds(j * 16, 16)] = j * 16 + lane
            dai_v[pl.ds(j * 16, 16)] = jnp.full((16,), -1, jnp.int32)
            return 0

        lax.fori_loop(0, DLIST // 16, dinit_body, 0)

        def compact_half(h, carry):
            def cmp_body(i, carry):
                off, offd = carry
                x = rm_v[pl.ds(i * 16, 16)]
                u = _key(x)
                m = u >= t2

                def do_store(carry):
                    off, offd = carry
                    am = am_v[pl.ds(i * 16, 16)]
                    row = h * HALF + i * 16 + lane
                    fl = row * N_CLS + am
                    offc = jnp.minimum(off, CAND_CAP - 16)
                    plsc.store_compressed(sortk_v.at[pl.ds(offc, 16)], u, mask=m)
                    plsc.store_compressed(sortv_v.at[pl.ds(offc, 16)], fl, mask=m)
                    m2u = _key(m2_v[pl.ds(i * 16, 16)])
                    md = m & (m2u >= t2)
                    offdc = jnp.minimum(offd, DLIST - 16)
                    plsc.store_compressed(
                        dchunk_v.at[pl.ds(offdc, 16)], row, mask=md
                    )
                    plsc.store_compressed(dai_v.at[pl.ds(offdc, 16)], fl, mask=md)
                    return off + _popcount(m), offd + _popcount(md)

                return lax.cond(jnp.any(m), do_store, lambda cr: cr, carry)

            return lax.fori_loop(0, HV, cmp_body, carry)

        carry = compact_half(1, (jnp.int32(0), jnp.int32(0)))
        stage_half(0)
        n_cand, n_deep = compact_half(0, carry)
        n_deep = jnp.minimum(n_deep, DEEP_CAP)

        def deep_scan(dc, off):
            zero16 = jnp.zeros((16,), jnp.int32)
            r_vec = plsc.load_gather(dchunk_v, [zero16 + dc])
            ai = plsc.load_gather(dai_v, [zero16 + dc])
            r_s = jnp.max(r_vec)
            r128 = jnp.minimum((r_s >> 7) << 7, N_ROWS - 128)
            col = r_s - r128
            src0 = pl.multiple_of(r128, 128)
            pltpu.sync_copy(lt_hbm.at[:, b, pl.ds(src0, 128)], deepstage_v)

            def kb(k, off):
                cls = k * 16 + lane
                v = plsc.load_gather(
                    deepstage_v, [jnp.minimum(cls, N_CLS), zero16 + col]
                )
                u = _key(v)
                fl = r_vec * N_CLS + cls
                m = (cls < N_CLS) & (u >= t2) & (fl != ai)

                def dstore(off):
                    offc = jnp.minimum(off, CAND_CAP - 16)
                    plsc.store_compressed(sortk_v.at[pl.ds(offc, 16)], u, mask=m)
                    plsc.store_compressed(sortv_v.at[pl.ds(offc, 16)], fl, mask=m)
                    return off + _popcount(m)

                return lax.cond(jnp.any(m), dstore, lambda o: o, off)

            return lax.fori_loop(0, 6, kb, off)

        n_cand = lax.fori_loop(0, n_deep, deep_scan, n_cand)

        nv = CAND_CAP // 16

        def inter_stage(ksz, j):
            jb = j // 16
            s = int(math.log2(jb)) if jb > 0 else 0

            def pair_body(t, _):
                v = ((t >> s) << (s + 1)) | (t & (jb - 1))
                p = v | jb
                ka = sortk_v[pl.ds(v * 16, 16)]
                va = sortv_v[pl.ds(v * 16, 16)]
                kb_ = sortk_v[pl.ds(p * 16, 16)]
                vb = sortv_v[pl.ds(p * 16, 16)]
                dir_asc = ((v * 16) & ksz) == 0
                lo_before = (ka > kb_) | ((ka == kb_) & (va < vb))
                swap = lo_before ^ dir_asc
                sortk_v[pl.ds(v * 16, 16)] = jnp.where(swap, kb_, ka)
                sortv_v[pl.ds(v * 16, 16)] = jnp.where(swap, vb, va)
                sortk_v[pl.ds(p * 16, 16)] = jnp.where(swap, ka, kb_)
                sortv_v[pl.ds(p * 16, 16)] = jnp.where(swap, va, vb)
                return 0

            lax.fori_loop(0, nv // 2, pair_body, 0)

        def intra_stage(ksz, j):
            pidx = lane ^ j

            def vreg_body(v, _):
                ka = sortk_v[pl.ds(v * 16, 16)]
                va = sortv_v[pl.ds(v * 16, 16)]
                kb_ = _shuf(ka, pidx)
                vb = _shuf(va, pidx)
                am_lower = (lane & j) == 0
                klo = jnp.where(am_lower, ka, kb_)
                khi = jnp.where(am_lower, kb_, ka)
                vlo = jnp.where(am_lower, va, vb)
                vhi = jnp.where(am_lower, vb, va)
                dir_asc = (((v * 16 + lane) & ksz) == 0)
                lo_before = (klo > khi) | ((klo == khi) & (vlo < vhi))
                swap = lo_before ^ dir_asc
                sortk_v[pl.ds(v * 16, 16)] = jnp.where(swap, kb_, ka)
                sortv_v[pl.ds(v * 16, 16)] = jnp.where(swap, vb, va)
                return 0

            lax.fori_loop(0, nv, vreg_body, 0)

        ksz = 2
        while ksz <= CAND_CAP:
            j = ksz // 2
            while j >= 1:
                if j >= 16:
                    inter_stage(ksz, j)
                else:
                    intra_stage(ksz, j)
                j //= 2
            ksz *= 2

        def out_body(jv, _):
            sl = pl.ds(jv * 16, 16)
            u = sortk_v[sl]
            fl = sortv_v[sl]
            x = _unkey(u)
            scores_v[sl] = 1.0 / (1.0 + jnp.exp(-x))
            br = _fdiv(fl, N_CLS)
            labels_v[sl] = fl - br * N_CLS
            boxidx_v[sl] = br
            return 0

        lax.fori_loop(0, OUT_W // 16, out_body, 0)

        pltpu.sync_copy(scores_v, scores_out.at[b])
        pltpu.sync_copy(labels_v, labels_out.at[b])

        pltpu.sync_copy(boxes_hbm.at[b], boxstage_v)
        sc_vec = _shuf(ts_v[pl.ds(0, 16)], (lane & 1) ^ 1)

        def box_body(jv, _):
            pos = jv * 16 + lane
            q = pos & 3
            br = plsc.load_gather(boxidx_v, [pos >> 2])
            v = plsc.load_gather(boxstage_v, [q, br])
            vp = plsc.load_gather(boxstage_v, [q ^ 2, br])
            xy = jnp.where(q < 2, v - 0.5 * vp, vp + 0.5 * v)
            boxout_v[pl.ds(jv * 16, 16)] = xy * sc_vec
            return 0

        lax.fori_loop(0, OUT_W * 4 // 16, box_body, 0)
        pltpu.sync_copy(boxout_v, boxes_out.at[b])


def _run_sc(rm, am, m2, lt, boxes_t, ts_pad):
    mesh = plsc.VectorSubcoreMesh(core_axis_name="c", subcore_axis_name="s")
    f = pl.kernel(
        _sc_body,
        mesh=mesh,
        compiler_params=pltpu.CompilerParams(needs_layout_passes=False),
        out_type=[
            jax.ShapeDtypeStruct((16, OUT_W), jnp.float32),
            jax.ShapeDtypeStruct((16, OUT_W), jnp.int32),
            jax.ShapeDtypeStruct((16, OUT_W * 4), jnp.float32),
        ],
        scratch_types=[
            pltpu.VMEM((HALF,), jnp.float32),
            pltpu.VMEM((HALF,), jnp.int32),
            pltpu.VMEM((HALF,), jnp.float32),
            pltpu.VMEM((BV * 16,), jnp.uint32),
            pltpu.VMEM((CAND_CAP,), jnp.uint32),
            pltpu.VMEM((CAND_CAP,), jnp.int32),
            pltpu.VMEM((DLIST,), jnp.int32),
            pltpu.VMEM((DLIST,), jnp.int32),
            pltpu.VMEM((92, 128), jnp.float32),
            pltpu.VMEM((4, N_ROWS), jnp.float32),
            pltpu.VMEM((OUT_W,), jnp.int32),
            pltpu.VMEM((OUT_W,), jnp.float32),
            pltpu.VMEM((OUT_W,), jnp.int32),
            pltpu.VMEM((OUT_W * 4,), jnp.float32),
            pltpu.VMEM((16,), jnp.float32),
            pltpu.SemaphoreType.DMA,
        ],
    )
    return f(rm, am, m2, lt, boxes_t, ts_pad)


def kernel(pred_logits, pred_boxes, target_sizes):
    bs, n, c = pred_logits.shape
    lt = jnp.transpose(pred_logits, (2, 0, 1))
    rm, am, m2 = pl.pallas_call(
        _rowstats_body,
        grid=(c // 4,),
        in_specs=[pl.BlockSpec((4, bs, n), lambda cc: (cc, 0, 0))],
        out_specs=[pl.BlockSpec((bs, NP), lambda cc: (0, 0))] * 3,
        out_shape=[
            jax.ShapeDtypeStruct((bs, NP), jnp.float32),
            jax.ShapeDtypeStruct((bs, NP), jnp.int32),
            jax.ShapeDtypeStruct((bs, NP), jnp.float32),
        ],
    )(lt)

    boxes_t = jnp.transpose(pred_boxes, (0, 2, 1))
    ts_pad = jnp.pad(target_sizes, ((0, 0), (0, 14)))

    scores_p, labels_p, boxes_p = _run_sc(rm, am, m2, lt, boxes_t, ts_pad)
    scores = scores_p[:, :300]
    labels = labels_p[:, :300]
    boxes = boxes_p.reshape(bs, OUT_W, 4)[:, :300]
    return scores, labels, boxes
